# baseline (speedup 1.0000x reference)
import functools

import jax
import jax.numpy as jnp
from jax import lax
from jax.experimental import pallas as pl
from jax.experimental.pallas import tpu as pltpu
from jax.experimental.pallas import tpu_sc as plsc

_VOCAB = 1000000
_EMBED = 64
_BATCH = 4096
_HIST = 50

_B_TOTAL = _BATCH * _HIST
_NC = 2
_NS = 16
_NW = _NC * _NS
_B_PER_W = _B_TOTAL // _NW
_CHUNK = 128
_N_CHUNKS = _B_PER_W // _CHUNK

_mesh = plsc.VectorSubcoreMesh(core_axis_name="c", subcore_axis_name="s")


@functools.partial(
    pl.kernel,
    mesh=_mesh,
    out_type=jax.ShapeDtypeStruct((_B_TOTAL, _EMBED), jnp.float32),
    scratch_types=[
        pltpu.VMEM((_N_CHUNKS, _CHUNK), jnp.int32),
        pltpu.VMEM((_CHUNK, _EMBED), jnp.float32),
        pltpu.SemaphoreType.DMA,
    ],
    compiler_params=pltpu.CompilerParams(use_tc_tiling_on_sc=False),
)
def _gather_kernel(idx_hbm, table_hbm, out_hbm, idx_v, rows_v, gsem):
    wid = lax.axis_index("s") * _NC + lax.axis_index("c")
    pltpu.sync_copy(idx_hbm.at[wid], idx_v)

    def body(c, carry):
        pltpu.async_copy(table_hbm.at[idx_v.at[c]], rows_v, gsem).wait()
        base = wid * _B_PER_W + c * _CHUNK
        pltpu.sync_copy(rows_v, out_hbm.at[pl.ds(base, _CHUNK)])
        return carry

    lax.fori_loop(0, _N_CHUNKS, body, 0)


def kernel(indices, table):
    idx = indices.astype(jnp.int32).reshape(_NW, _N_CHUNKS, _CHUNK)
    out = _gather_kernel(idx, table)
    return out.reshape(_BATCH, _HIST, _EMBED)


# speedup vs baseline: 1.0403x; 1.0403x over previous
import functools

import jax
import jax.numpy as jnp
from jax import lax
from jax.experimental import pallas as pl
from jax.experimental.pallas import tpu as pltpu
from jax.experimental.pallas import tpu_sc as plsc

_VOCAB = 1000000
_EMBED = 64
_BATCH = 4096
_HIST = 50

_B_TOTAL = _BATCH * _HIST
_NC = 2
_NS = 16
_NW = _NC * _NS
_B_PER_W = _B_TOTAL // _NW
_CHUNK = 128
_N_CHUNKS = _B_PER_W // _CHUNK
_K = 5
_N_GROUPS = _N_CHUNKS // _K

_mesh = plsc.VectorSubcoreMesh(core_axis_name="c", subcore_axis_name="s")


@functools.partial(
    pl.kernel,
    mesh=_mesh,
    out_type=jax.ShapeDtypeStruct((_B_TOTAL, _EMBED), jnp.float32),
    scratch_types=[
        pltpu.VMEM((_N_CHUNKS, _CHUNK), jnp.int32),
        pltpu.VMEM((_K, _CHUNK, _EMBED), jnp.float32),
        pltpu.VMEM((_K, _CHUNK, _EMBED), jnp.float32),
        pltpu.SemaphoreType.DMA,
        pltpu.SemaphoreType.DMA,
        pltpu.SemaphoreType.DMA,
        pltpu.SemaphoreType.DMA,
    ],
    compiler_params=pltpu.CompilerParams(use_tc_tiling_on_sc=False),
)
def _gather_kernel(idx_hbm, table_hbm, out_hbm, idx_v, bufs_a, bufs_b,
                   gsem_a, gsem_b, osem_a, osem_b):
    wid = lax.axis_index("s") * _NC + lax.axis_index("c")
    pltpu.sync_copy(idx_hbm.at[wid], idx_v)
    out_base = wid * _B_PER_W

    def _gather(c, buf, sem):
        return pltpu.async_copy(table_hbm.at[idx_v.at[c]], buf, sem)

    def _writeback(c, buf, sem):
        return pltpu.async_copy(
            buf, out_hbm.at[pl.ds(out_base + c * _CHUNK, _CHUNK)], sem)

    def _drain_writes(buf, dst0, sem):
        pltpu.make_async_copy(out_hbm.at[pl.ds(dst0, _CHUNK)], buf, sem).wait()

    def body(g, carry):
        ca = 2 * _K * g
        cb = ca + _K

        @pl.when(g > 0)
        def _():
            for b in range(_K):
                _drain_writes(bufs_a.at[b], out_base, osem_a)

        ga = [_gather(ca + b, bufs_a.at[b], gsem_a) for b in range(_K)]

        @pl.when(g > 0)
        def _():
            for b in range(_K):
                _drain_writes(bufs_b.at[b], out_base, osem_b)

        for d in ga:
            d.wait()
        for b in range(_K):
            _writeback(ca + b, bufs_a.at[b], osem_a)

        gb = [_gather(cb + b, bufs_b.at[b], gsem_b) for b in range(_K)]
        for d in gb:
            d.wait()
        for b in range(_K):
            _writeback(cb + b, bufs_b.at[b], osem_b)
        return carry

    lax.fori_loop(0, _N_GROUPS // 2, body, 0)
    for b in range(_K):
        _drain_writes(bufs_a.at[b], out_base, osem_a)
        _drain_writes(bufs_b.at[b], out_base, osem_b)


def kernel(indices, table):
    idx = indices.astype(jnp.int32).reshape(_NW, _N_CHUNKS, _CHUNK)
    out = _gather_kernel(idx, table)
    return out.reshape(_BATCH, _HIST, _EMBED)


# speedup vs baseline: 1.0427x; 1.0023x over previous
import functools

import jax
import jax.numpy as jnp
from jax import lax
from jax.experimental import pallas as pl
from jax.experimental.pallas import tpu as pltpu
from jax.experimental.pallas import tpu_sc as plsc

_VOCAB = 1000000
_EMBED = 64
_BATCH = 4096
_HIST = 50

_B_TOTAL = _BATCH * _HIST
_NC = 2
_NS = 16
_NW = _NC * _NS
_B_PER_W = _B_TOTAL // _NW
_CHUNK = 640
_N_CHUNKS = _B_PER_W // _CHUNK
_N_PAIRS = _N_CHUNKS // 2

_mesh = plsc.VectorSubcoreMesh(core_axis_name="c", subcore_axis_name="s")


@functools.partial(
    pl.kernel,
    mesh=_mesh,
    out_type=jax.ShapeDtypeStruct((_B_TOTAL, _EMBED), jnp.float32),
    scratch_types=[
        pltpu.VMEM((_N_CHUNKS, _CHUNK), jnp.int32),
        pltpu.VMEM((_CHUNK, _EMBED), jnp.float32),
        pltpu.VMEM((_CHUNK, _EMBED), jnp.float32),
        pltpu.SemaphoreType.DMA,
        pltpu.SemaphoreType.DMA,
        pltpu.SemaphoreType.DMA,
        pltpu.SemaphoreType.DMA,
    ],
    compiler_params=pltpu.CompilerParams(use_tc_tiling_on_sc=False),
)
def _gather_kernel(idx_hbm, table_hbm, out_hbm, idx_v, buf_a, buf_b,
                   gsem_a, gsem_b, osem_a, osem_b):
    wid = lax.axis_index("s") * _NC + lax.axis_index("c")
    pltpu.sync_copy(idx_hbm.at[wid], idx_v)
    out_base = wid * _B_PER_W

    def _gather(c, buf, sem):
        return pltpu.async_copy(table_hbm.at[idx_v.at[c]], buf, sem)

    def _writeback(c, buf, sem):
        return pltpu.async_copy(
            buf, out_hbm.at[pl.ds(out_base + c * _CHUNK, _CHUNK)], sem)

    def _drain_write(buf, sem):
        pltpu.make_async_copy(out_hbm.at[pl.ds(out_base, _CHUNK)], buf,
                              sem).wait()

    def body(g, carry):
        ca = 2 * g
        cb = ca + 1

        ga = _gather(ca, buf_a, gsem_a)

        @pl.when(g > 0)
        def _():
            _drain_write(buf_b, osem_b)

        ga.wait()
        _writeback(ca, buf_a, osem_a)
        gb = _gather(cb, buf_b, gsem_b)

        @pl.when(g < _N_PAIRS - 1)
        def _():
            _drain_write(buf_a, osem_a)

        gb.wait()
        _writeback(cb, buf_b, osem_b)
        return carry

    lax.fori_loop(0, _N_PAIRS, body, 0)
    _drain_write(buf_a, osem_a)
    _drain_write(buf_b, osem_b)


def kernel(indices, table):
    idx = indices.astype(jnp.int32).reshape(_NW, _N_CHUNKS, _CHUNK)
    out = _gather_kernel(idx, table)
    return out.reshape(_BATCH, _HIST, _EMBED)


# speedup vs baseline: 1.0589x; 1.0155x over previous
import functools

import jax
import jax.numpy as jnp
from jax import lax
from jax.experimental import pallas as pl
from jax.experimental.pallas import tpu as pltpu
from jax.experimental.pallas import tpu_sc as plsc

_VOCAB = 1000000
_EMBED = 64
_BATCH = 4096
_HIST = 50

_B_TOTAL = _BATCH * _HIST
_NC = 2
_NS = 16
_NW = _NC * _NS
_B_PER_W = _B_TOTAL // _NW
_CHUNK = 640
_N_CHUNKS = _B_PER_W // _CHUNK
_N_PAIRS = _N_CHUNKS // 2

_mesh = plsc.VectorSubcoreMesh(core_axis_name="c", subcore_axis_name="s")


@functools.partial(
    pl.kernel,
    mesh=_mesh,
    out_type=jax.ShapeDtypeStruct((_B_TOTAL, _EMBED), jnp.float32),
    scratch_types=[
        pltpu.VMEM((_N_CHUNKS, _CHUNK), jnp.int32),
        pltpu.VMEM((_CHUNK, _EMBED), jnp.float32),
        pltpu.VMEM((_CHUNK, _EMBED), jnp.float32),
        pltpu.SemaphoreType.DMA,
        pltpu.SemaphoreType.DMA,
        pltpu.SemaphoreType.DMA,
        pltpu.SemaphoreType.DMA,
    ],
    compiler_params=pltpu.CompilerParams(use_tc_tiling_on_sc=False),
)
def _gather_kernel(idx_hbm, table_hbm, out_hbm, idx_v, buf_a, buf_b,
                   gsem_a, gsem_b, osem_a, osem_b):
    wid = lax.axis_index("s") * _NC + lax.axis_index("c")
    pltpu.sync_copy(idx_hbm.at[wid], idx_v)
    out_base = wid * _B_PER_W

    def _gather(c, buf, sem):
        return pltpu.async_copy(table_hbm.at[idx_v.at[c]], buf, sem)

    def _writeback(c, buf, sem):
        return pltpu.async_copy(
            buf, out_hbm.at[pl.ds(out_base + c * _CHUNK, _CHUNK)], sem)

    def _drain_write(buf, sem):
        pltpu.make_async_copy(out_hbm.at[pl.ds(out_base, _CHUNK)], buf,
                              sem).wait()

    def body(g, carry):
        ca = 2 * g
        cb = ca + 1

        ga = _gather(ca, buf_a, gsem_a)

        @pl.when(g > 0)
        def _():
            _drain_write(buf_b, osem_b)

        ga.wait()
        _writeback(ca, buf_a, osem_a)
        gb = _gather(cb, buf_b, gsem_b)

        @pl.when(g < _N_PAIRS - 1)
        def _():
            _drain_write(buf_a, osem_a)

        gb.wait()
        _writeback(cb, buf_b, osem_b)
        return carry

    lax.fori_loop(0, _N_PAIRS, body, 0)
    _drain_write(buf_a, osem_a)
    _drain_write(buf_b, osem_b)


def kernel(indices, table):
    idx = indices.T.astype(jnp.int32).reshape(_NW, _N_CHUNKS, _CHUNK)
    out = _gather_kernel(idx, table)
    return out.reshape(_HIST, _BATCH, _EMBED).transpose(1, 0, 2)
